# baseline (speedup 1.0000x reference)
import functools

import jax
import jax.numpy as jnp
from jax import lax
from jax.experimental import pallas as pl
from jax.experimental.pallas import tpu as pltpu
from jax.experimental.pallas import tpu_sc as plsc


def _swap(v, k):
    idx = jax.lax.iota(jnp.int32, 16) ^ k
    dnums = lax.GatherDimensionNumbers(
        offset_dims=(), collapsed_slice_dims=(0,), start_index_map=(0,))
    return lax.gather(v, idx[:, None], dnums, slice_sizes=(1,),
                      mode=lax.GatherScatterMode.PROMISE_IN_BOUNDS)


def kernel(activation):
    B, C, H, W = activation.shape
    n_images = B * C
    mesh = plsc.VectorSubcoreMesh(core_axis_name="c", subcore_axis_name="s")
    info = plsc.get_sparse_core_info()
    nw = info.num_cores * info.num_subcores
    per_w = n_images // nw
    nchunk = W // 16

    @functools.partial(
        pl.kernel,
        mesh=mesh,
        out_type=jax.ShapeDtypeStruct((B, C, H, W), jnp.float32),
        scratch_types=[pltpu.VMEM((H, W), jnp.float32)],
    )
    def run(act, out, buf):
        wid = lax.axis_index("s") * info.num_cores + lax.axis_index("c")

        def do_relu(_):
            def row(r, carry):
                for c in range(nchunk):
                    sl = pl.ds(c * 16, 16)
                    v = buf[r, sl]
                    buf[r, sl] = jnp.where(v >= 0, v, 0.0)
                return carry
            lax.fori_loop(0, H, row, 0)
            return 0

        def do_b2(_):
            def row(r, carry):
                for c in range(nchunk):
                    sl = pl.ds(c * 16, 16)
                    v0 = buf[2 * r, sl]
                    v1 = buf[2 * r + 1, sl]
                    s = v0 + v1
                    s = s + _swap(s, 1)
                    m = s >= 0
                    buf[2 * r, sl] = jnp.where(m, v0, 0.0)
                    buf[2 * r + 1, sl] = jnp.where(m, v1, 0.0)
                return carry
            lax.fori_loop(0, H // 2, row, 0)
            return 0

        def do_b4(_):
            def row(r, carry):
                for c in range(nchunk):
                    sl = pl.ds(c * 16, 16)
                    v0 = buf[4 * r, sl]
                    v1 = buf[4 * r + 1, sl]
                    v2 = buf[4 * r + 2, sl]
                    v3 = buf[4 * r + 3, sl]
                    s = (v0 + v1) + (v2 + v3)
                    s = s + _swap(s, 1)
                    s = s + _swap(s, 2)
                    m = s >= 0
                    buf[4 * r, sl] = jnp.where(m, v0, 0.0)
                    buf[4 * r + 1, sl] = jnp.where(m, v1, 0.0)
                    buf[4 * r + 2, sl] = jnp.where(m, v2, 0.0)
                    buf[4 * r + 3, sl] = jnp.where(m, v3, 0.0)
                return carry
            lax.fori_loop(0, H // 4, row, 0)
            return 0

        def image(i, carry):
            n = wid * per_w + i
            b = n // C
            ch = n % C
            g = ch // 48

            @pl.when(g == 3)
            def _():
                pltpu.sync_copy(act.at[b, ch], out.at[b, ch])

            @pl.when(g != 3)
            def _():
                pltpu.sync_copy(act.at[b, ch], buf)
                lax.switch(g, [do_relu, do_b2, do_b4], 0)
                pltpu.sync_copy(buf, out.at[b, ch])

            return carry

        lax.fori_loop(0, per_w, image, 0)

    return run(activation)


# speedup vs baseline: 10.6315x; 10.6315x over previous
import jax
import jax.numpy as jnp
from jax.experimental import pallas as pl
from jax.experimental.pallas import tpu as pltpu

_CB = 48


def _roll(x, shift, axis):
    return pltpu.roll(x, shift % x.shape[axis], axis)


def _pair_sum_bcast(x, axis, b):
    idx = jax.lax.broadcasted_iota(jnp.int32, x.shape, axis)
    if b == 2:
        s = x + _roll(x, -1, axis)
        return jnp.where(idx % 2 == 0, s, _roll(s, 1, axis))
    s1 = x + _roll(x, -1, axis)
    s = s1 + _roll(s1, -2, axis)
    t = jnp.where(idx % 2 == 0, s, _roll(s, 1, axis))
    return jnp.where(idx % 4 < 2, t, _roll(t, 2, axis))


def _body(x_ref, o_ref):
    cc = pl.program_id(1)
    g = cc // (48 // _CB)

    @pl.when(g == 0)
    def _():
        x = x_ref[...]
        o_ref[...] = x * (x >= 0).astype(x.dtype)

    @pl.when(g == 3)
    def _():
        o_ref[...] = x_ref[...]

    def block_group(b, lanes_on_mxu):
        c, h, w = x_ref.shape[1], x_ref.shape[2], x_ref.shape[3]
        x = x_ref[...].reshape(c * h, w)
        if lanes_on_mxu:
            i = jax.lax.broadcasted_iota(jnp.int32, (w, w), 0)
            j = jax.lax.broadcasted_iota(jnp.int32, (w, w), 1)
            e = (i // b == j // b).astype(jnp.bfloat16)
            hi = x.astype(jnp.bfloat16)
            lo = (x - hi.astype(jnp.float32)).astype(jnp.bfloat16)
            f32 = jnp.float32
            s = (jax.lax.dot_general(hi, e, (((1,), (0,)), ((), ())),
                                     preferred_element_type=f32)
                 + jax.lax.dot_general(lo, e, (((1,), (0,)), ((), ())),
                                       preferred_element_type=f32))
        else:
            s = _pair_sum_bcast(x, 1, b)
        s = _pair_sum_bcast(s, 0, b)
        o_ref[...] = (x * (s >= 0).astype(x.dtype)).reshape(1, c, h, w)

    @pl.when(g == 1)
    def _():
        block_group(2, lanes_on_mxu=True)

    @pl.when(g == 2)
    def _():
        block_group(4, lanes_on_mxu=True)


def kernel(activation):
    B, C, H, W = activation.shape
    return pl.pallas_call(
        _body,
        grid=(B, C // _CB, 2),
        in_specs=[pl.BlockSpec((1, _CB, H // 2, W),
                               lambda b, c, h: (b, c, h, 0))],
        out_specs=pl.BlockSpec((1, _CB, H // 2, W),
                               lambda b, c, h: (b, c, h, 0)),
        out_shape=jax.ShapeDtypeStruct(activation.shape, activation.dtype),
        compiler_params=pltpu.CompilerParams(
            dimension_semantics=("parallel", "parallel", "parallel")),
    )(activation)


# speedup vs baseline: 11.3904x; 1.0714x over previous
import jax
import jax.numpy as jnp
from jax.experimental import pallas as pl
from jax.experimental.pallas import tpu as pltpu

_CB = 48


def _roll(x, shift, axis):
    return pltpu.roll(x, shift % x.shape[axis], axis)


def _pair_sum_bcast(x, axis, b):
    idx = jax.lax.broadcasted_iota(jnp.int32, x.shape, axis)
    if b == 2:
        s = x + _roll(x, -1, axis)
        return jnp.where(idx % 2 == 0, s, _roll(s, 1, axis))
    s1 = x + _roll(x, -1, axis)
    s = s1 + _roll(s1, -2, axis)
    t = jnp.where(idx % 2 == 0, s, _roll(s, 1, axis))
    return jnp.where(idx % 4 < 2, t, _roll(t, 2, axis))


def _body(x_ref, o_ref):
    cc = pl.program_id(1)
    g = cc // (48 // _CB)

    @pl.when(g == 0)
    def _():
        x = x_ref[...]
        o_ref[...] = x * (x >= 0).astype(x.dtype)

    @pl.when(g == 3)
    def _():
        o_ref[...] = x_ref[...]

    def block_group(b, lanes_on_mxu):
        c, h, w = x_ref.shape[1], x_ref.shape[2], x_ref.shape[3]
        x = x_ref[...].reshape(c * h, w)
        if lanes_on_mxu:
            i = jax.lax.broadcasted_iota(jnp.int32, (w, w), 0)
            j = jax.lax.broadcasted_iota(jnp.int32, (w, w), 1)
            e = (i // b == j // b).astype(jnp.bfloat16)
            hi = x.astype(jnp.bfloat16)
            lo = (x - hi.astype(jnp.float32)).astype(jnp.bfloat16)
            f32 = jnp.float32
            s = (jax.lax.dot_general(hi, e, (((1,), (0,)), ((), ())),
                                     preferred_element_type=f32)
                 + jax.lax.dot_general(lo, e, (((1,), (0,)), ((), ())),
                                       preferred_element_type=f32))
        else:
            s = _pair_sum_bcast(x, 1, b)
        s = _pair_sum_bcast(s, 0, b)
        o_ref[...] = (x * (s >= 0).astype(x.dtype)).reshape(1, c, h, w)

    @pl.when(g == 1)
    def _():
        block_group(2, lanes_on_mxu=True)

    @pl.when(g == 2)
    def _():
        block_group(4, lanes_on_mxu=True)


def kernel(activation):
    B, C, H, W = activation.shape
    return pl.pallas_call(
        _body,
        grid=(B, C // _CB),
        in_specs=[pl.BlockSpec((1, _CB, H, W), lambda b, c: (b, c, 0, 0))],
        out_specs=pl.BlockSpec((1, _CB, H, W), lambda b, c: (b, c, 0, 0)),
        out_shape=jax.ShapeDtypeStruct(activation.shape, activation.dtype),
        compiler_params=pltpu.CompilerParams(
            dimension_semantics=("parallel", "parallel")),
    )(activation)
